# Initial kernel scaffold; baseline (speedup 1.0000x reference)
import jax
import jax.numpy as jnp
from jax.experimental import pallas as pl


def kernel(x, struct_embed, W_qkv, W_out, b_out, ln1_g, ln1_b, ln2_g, ln2_b, W_route, b_route, W_noise, b_noise, We1, be1, We2, be2, idx):
    raise NotImplementedError("write your pallas kernel here")


# speedup vs baseline: 1.1901x; 1.1901x over previous
import functools

import jax
import jax.numpy as jnp
from jax.experimental import pallas as pl
from jax.experimental.pallas import tpu as pltpu

F32 = jnp.float32
T = 2048
C = 768
NH = 12
HD = 64
NE = 8
DFF = 3072
TB = 256
QB = 512
NTB = T // TB
NQB = T // QB


def _ln(x, g, b):
    mu = jnp.mean(x, axis=-1, keepdims=True)
    var = jnp.mean((x - mu) ** 2, axis=-1, keepdims=True)
    return (x - mu) / jnp.sqrt(var + 1e-5) * g + b


def _prologue_kernel(x_ref, oh_ref, se_ref, g_ref, b_ref, h_ref):
    x = x_ref[...]
    h = _ln(x, g_ref[...], b_ref[...])
    h_ref[...] = h + jnp.dot(oh_ref[...], se_ref[...],
                             preferred_element_type=F32)


def _qkv_kernel(h_ref, w_ref, qkv_ref):
    qkv_ref[0] = jnp.dot(h_ref[...], w_ref[0], preferred_element_type=F32)


def _attn_kernel(q_ref, k_ref, v_ref, o_ref):
    q = q_ref[0]
    s = jax.lax.dot_general(q, k_ref[0], (((1,), (1,)), ((), ())),
                            preferred_element_type=F32) * 0.125
    s = jnp.clip(s, -30.0, 30.0)
    mx = jnp.max(s, axis=-1, keepdims=True)
    p = jnp.exp(s - mx)
    denom = jnp.sum(p, axis=-1, keepdims=True)
    o_ref[0] = jnp.dot(p, v_ref[0], preferred_element_type=F32) / denom


def _post_kernel(x_ref, ctx_ref, wout_ref, bout_ref, g2_ref, b2_ref,
                 wr_ref, br_ref, wn_ref, bn_ref, nz_ref,
                 x2_ref, m_ref, gates_ref):
    x2 = x_ref[...] + jnp.dot(ctx_ref[...], wout_ref[...],
                              preferred_element_type=F32) + bout_ref[...]
    x2_ref[...] = x2
    m = _ln(x2, g2_ref[...], b2_ref[...])
    m_ref[...] = m
    logits = jnp.dot(m, wr_ref[...], preferred_element_type=F32) + br_ref[...]
    nl = jnp.dot(m, wn_ref[...], preferred_element_type=F32) + bn_ref[...]
    sp = jnp.maximum(nl, 0.0) + jnp.log1p(jnp.exp(-jnp.abs(nl)))
    noisy = logits + nz_ref[...] * sp
    v1 = jnp.max(noisy, axis=-1, keepdims=True)
    masked = jnp.where(noisy == v1, -jnp.inf, noisy)
    v2 = jnp.max(masked, axis=-1, keepdims=True)
    sel = noisy >= v2
    e1 = jnp.exp(noisy - v1)
    gates_un = jnp.where(sel, e1, 0.0)
    gates_ref[...] = gates_un / jnp.sum(gates_un, axis=-1, keepdims=True)


DC = 768
ND = DFF // DC


def _moe_dense_kernel(x2_ref, m_ref, gates_ref, w1_ref, b1_ref, w2_ref,
                      b2_ref, out_ref):
    e = pl.program_id(0)
    d = pl.program_id(1)
    m = m_ref[...]
    h = jnp.dot(m, w1_ref[0], preferred_element_type=F32) + b1_ref[0]
    h = jax.nn.gelu(h)
    part = jnp.dot(h, w2_ref[0], preferred_element_type=F32)
    onehot = (jax.lax.broadcasted_iota(jnp.int32, (1, NE), 1) == e).astype(F32)
    gate = jnp.sum(gates_ref[...] * onehot, axis=1, keepdims=True)

    @pl.when((e == 0) & (d == 0))
    def _():
        out_ref[...] = x2_ref[...]

    @pl.when(d == 0)
    def _():
        out_ref[...] += gate * b2_ref[0]

    out_ref[...] += gate * part


def _full(shape):
    return pl.BlockSpec(shape, lambda *_: tuple(0 for _ in shape))


def kernel(x, struct_embed, W_qkv, W_out, b_out, ln1_g, ln1_b, ln2_g, ln2_b,
           W_route, b_route, W_noise, b_noise, We1, be1, We2, be2, idx):
    B_, T_, C_ = x.shape
    xf = x.reshape(T_, C_)
    ids = idx.reshape(T_)
    sid = ((ids == 1) * 1 + (ids == 2) * 2 + (ids == 3) * 3).astype(jnp.int32)
    onehot = (sid[:, None] == jnp.arange(4)[None, :]).astype(F32)
    noise = jax.random.normal(jax.random.key(42), (T_, NE), dtype=F32)

    ln1_g2, ln1_b2 = ln1_g.reshape(1, C), ln1_b.reshape(1, C)
    ln2_g2, ln2_b2 = ln2_g.reshape(1, C), ln2_b.reshape(1, C)
    b_out2 = b_out.reshape(1, C)
    b_route2, b_noise2 = b_route.reshape(1, NE), b_noise.reshape(1, NE)

    h = pl.pallas_call(
        _prologue_kernel,
        grid=(NTB,),
        in_specs=[
            pl.BlockSpec((TB, C), lambda i: (i, 0)),
            pl.BlockSpec((TB, 4), lambda i: (i, 0)),
            _full((4, C)),
            _full((1, C)),
            _full((1, C)),
        ],
        out_specs=pl.BlockSpec((TB, C), lambda i: (i, 0)),
        out_shape=jax.ShapeDtypeStruct((T, C), F32),
    )(xf, onehot, struct_embed, ln1_g2, ln1_b2)

    Wh = W_qkv.reshape(C, 3 * NH, HD).transpose(1, 0, 2)
    qkv3 = pl.pallas_call(
        _qkv_kernel,
        grid=(3 * NH,),
        in_specs=[
            _full((T, C)),
            pl.BlockSpec((1, C, HD), lambda j: (j, 0, 0)),
        ],
        out_specs=pl.BlockSpec((1, T, HD), lambda j: (j, 0, 0)),
        out_shape=jax.ShapeDtypeStruct((3 * NH, T, HD), F32),
    )(h, Wh)

    ctx3 = pl.pallas_call(
        _attn_kernel,
        grid=(NH, NQB),
        in_specs=[
            pl.BlockSpec((1, QB, HD), lambda h, qb: (h, qb, 0)),
            pl.BlockSpec((1, T, HD), lambda h, qb: (NH + h, 0, 0)),
            pl.BlockSpec((1, T, HD), lambda h, qb: (2 * NH + h, 0, 0)),
        ],
        out_specs=pl.BlockSpec((1, QB, HD), lambda h, qb: (h, qb, 0)),
        out_shape=jax.ShapeDtypeStruct((NH, T, HD), F32),
    )(qkv3, qkv3, qkv3)
    ctx = ctx3.transpose(1, 0, 2).reshape(T, C)

    x2, m, gates = pl.pallas_call(
        _post_kernel,
        grid=(NTB,),
        in_specs=[
            pl.BlockSpec((TB, C), lambda i: (i, 0)),
            pl.BlockSpec((TB, C), lambda i: (i, 0)),
            _full((C, C)),
            _full((1, C)),
            _full((1, C)),
            _full((1, C)),
            _full((C, NE)),
            _full((1, NE)),
            _full((C, NE)),
            _full((1, NE)),
            pl.BlockSpec((TB, NE), lambda i: (i, 0)),
        ],
        out_specs=[
            pl.BlockSpec((TB, C), lambda i: (i, 0)),
            pl.BlockSpec((TB, C), lambda i: (i, 0)),
            pl.BlockSpec((TB, NE), lambda i: (i, 0)),
        ],
        out_shape=[
            jax.ShapeDtypeStruct((T, C), F32),
            jax.ShapeDtypeStruct((T, C), F32),
            jax.ShapeDtypeStruct((T, NE), F32),
        ],
    )(xf, ctx, W_out, b_out2, ln2_g2, ln2_b2, W_route, b_route2,
      W_noise, b_noise2, noise)

    out = pl.pallas_call(
        _moe_dense_kernel,
        grid=(NE, ND),
        in_specs=[
            pl.BlockSpec((T, C), lambda e, d: (0, 0)),
            pl.BlockSpec((T, C), lambda e, d: (0, 0)),
            pl.BlockSpec((T, NE), lambda e, d: (0, 0)),
            pl.BlockSpec((1, C, DC), lambda e, d: (e, 0, d)),
            pl.BlockSpec((1, 1, DC), lambda e, d: (e, 0, d)),
            pl.BlockSpec((1, DC, C), lambda e, d: (e, d, 0)),
            pl.BlockSpec((1, 1, C), lambda e, d: (e, 0, 0)),
        ],
        out_specs=pl.BlockSpec((T, C), lambda e, d: (0, 0)),
        out_shape=jax.ShapeDtypeStruct((T, C), F32),
        compiler_params=pltpu.CompilerParams(
            dimension_semantics=("arbitrary", "arbitrary")),
    )(x2, m, gates, We1, be1.reshape(NE, 1, DFF), We2, be2.reshape(NE, 1, C))

    return out.reshape(B_, T_, C_)


# speedup vs baseline: 1.3874x; 1.1657x over previous
import functools

import jax
import jax.numpy as jnp
from jax import lax
from jax.experimental import pallas as pl
from jax.experimental.pallas import tpu as pltpu
from jax.experimental.pallas import tpu_sc as plsc

F32 = jnp.float32
I32 = jnp.int32
T = 2048
C = 768
NH = 12
HD = 64
NE = 8
DFF = 3072
TB = 256
QB = 512
NTB = T // TB
NQB = T // QB
A = 2 * T
G = 256
NB = A // G + NE
S = NB * G

NSC = 2
NTILE = 16
NW = NSC * NTILE


def _ln(x, g, b):
    mu = jnp.mean(x, axis=-1, keepdims=True)
    var = jnp.mean((x - mu) ** 2, axis=-1, keepdims=True)
    return (x - mu) / jnp.sqrt(var + 1e-5) * g + b


def _prologue_kernel(x_ref, oh_ref, se_ref, g_ref, b_ref, h_ref):
    x = x_ref[...]
    h = _ln(x, g_ref[...], b_ref[...])
    h_ref[...] = h + jnp.dot(oh_ref[...], se_ref[...],
                             preferred_element_type=F32)


def _qkv_kernel(h_ref, w_ref, qkv_ref):
    qkv_ref[0] = jnp.dot(h_ref[...], w_ref[0], preferred_element_type=F32)


def _attn_kernel(q_ref, k_ref, v_ref, o_ref):
    q = q_ref[0]
    s = lax.dot_general(q, k_ref[0], (((1,), (1,)), ((), ())),
                        preferred_element_type=F32) * 0.125
    s = jnp.clip(s, -30.0, 30.0)
    mx = jnp.max(s, axis=-1, keepdims=True)
    p = jnp.exp(s - mx)
    denom = jnp.sum(p, axis=-1, keepdims=True)
    o_ref[0] = jnp.dot(p, v_ref[0], preferred_element_type=F32) / denom


def _post_kernel(x_ref, ctx_ref, wout_ref, bout_ref, g2_ref, b2_ref,
                 wr_ref, br_ref, wn_ref, bn_ref, nz_ref,
                 x2_ref, m_ref, noisy_ref):
    x2 = x_ref[...] + jnp.dot(ctx_ref[...], wout_ref[...],
                              preferred_element_type=F32) + bout_ref[...]
    x2_ref[...] = x2
    m = _ln(x2, g2_ref[...], b2_ref[...])
    m_ref[...] = m
    logits = jnp.dot(m, wr_ref[...], preferred_element_type=F32) + br_ref[...]
    nl = jnp.dot(m, wn_ref[...], preferred_element_type=F32) + bn_ref[...]
    sp = jnp.maximum(nl, 0.0) + jnp.log1p(jnp.exp(-jnp.abs(nl)))
    noisy_ref[...] = logits + nz_ref[...] * sp


def _cumsum0(x):
    n = x.shape[0]
    k = 1
    while k < n:
        pad = jnp.zeros((k, x.shape[1]), x.dtype)
        x = x + jnp.concatenate([pad, x[:-k]], axis=0)
        k *= 2
    return x


def _route_kernel(noisy_ref, pos_ref, gv_ref, be_ref, nact_ref):
    noisy = noisy_ref[...]
    iota8 = lax.broadcasted_iota(I32, (T, NE), 1)
    v1 = jnp.max(noisy, axis=1, keepdims=True)
    is1 = noisy == v1
    e1 = jnp.min(jnp.where(is1, iota8, NE), axis=1, keepdims=True)
    masked = jnp.where(iota8 == e1, -jnp.inf, noisy)
    v2 = jnp.max(masked, axis=1, keepdims=True)
    is2 = masked == v2
    e2 = jnp.min(jnp.where(is2, iota8, NE), axis=1, keepdims=True)

    d = jnp.exp(v2 - v1)
    g1 = 1.0 / (1.0 + d)
    g2 = d / (1.0 + d)

    oh1 = (iota8 == e1).astype(I32)
    oh2 = (iota8 == e2).astype(I32)
    oh = jnp.concatenate([oh1, oh2], axis=0)
    cum = _cumsum0(oh)
    rank = jnp.sum(cum * oh, axis=1, keepdims=True) - 1

    counts = cum[A - 1:A, :]
    pcnt = ((counts + (G - 1)) // G) * G
    inc = pcnt
    for k in (1, 2, 4):
        inc = inc + jnp.concatenate(
            [jnp.zeros((1, k), I32), inc[:, :-k]], axis=1)
    poff = inc - pcnt

    po = jnp.sum(oh * poff, axis=1, keepdims=True)
    pos_ref[...] = po + rank

    g_all = jnp.concatenate([g1, g2], axis=0)
    gv_ref[...] = jnp.broadcast_to(g_all, (A, 128))

    iob = lax.broadcasted_iota(I32, (NB, NE), 0) * G
    be_ref[...] = jnp.sum((iob >= poff).astype(I32), axis=1,
                          keepdims=True) - 1
    nact_ref[...] = jnp.sum(pcnt, axis=1, keepdims=True) // G


def _ffn_kernel(be_ref, nact_ref, xs_ref, w1_ref, b1_ref, w2_ref, b2_ref,
                gs_ref, ys_ref):
    b = pl.program_id(0)

    @pl.when(b < nact_ref[0])
    def _():
        xb = xs_ref[...]
        h = jnp.dot(xb, w1_ref[0], preferred_element_type=F32) + b1_ref[0]
        h = jax.nn.gelu(h)
        y = jnp.dot(h, w2_ref[0], preferred_element_type=F32) + b2_ref[0]
        gate = gs_ref[...][:, :1]
        ys_ref[...] = gate * y


def _dispatch_body(m_hbm, pos_hbm, gv_hbm, xs_hbm, gs_hbm,
                   idx_v, rows_v, gvr_v, sem):
    c = lax.axis_index("c")
    s = lax.axis_index("s")
    w = s * NSC + c
    tok = lax.rem(w, NTILE) * (T // NTILE)
    pltpu.sync_copy(pos_hbm.at[w], idx_v)
    pltpu.sync_copy(m_hbm.at[pl.ds(tok, A // NW)], rows_v)
    pltpu.sync_copy(gv_hbm.at[pl.ds(w * (A // NW), A // NW)], gvr_v)
    pltpu.async_copy(rows_v, xs_hbm.at[idx_v], sem).wait()
    pltpu.async_copy(gvr_v, gs_hbm.at[idx_v], sem).wait()


def _combine_body(x2_hbm, pos_hbm, ys_hbm, out_hbm,
                  i1_v, i2_v, y1_v, y2_v, o_v, sem):
    c = lax.axis_index("c")
    s = lax.axis_index("s")
    w = s * NSC + c
    CH = 32
    for half in range(2):
        base = w * 64 + half * CH
        pltpu.sync_copy(pos_hbm.at[pl.ds(base, CH)], i1_v)
        pltpu.sync_copy(pos_hbm.at[pl.ds(T + base, CH)], i2_v)
        pltpu.async_copy(ys_hbm.at[i1_v], y1_v, sem).wait()
        pltpu.async_copy(ys_hbm.at[i2_v], y2_v, sem).wait()
        pltpu.sync_copy(x2_hbm.at[pl.ds(base, CH)], o_v)

        def body(i, _):
            def inner(j, _):
                sl = pl.ds(j * 16, 16)
                o_v[i, sl] = o_v[i, sl] + y1_v[i, sl] + y2_v[i, sl]
                return 0
            return lax.fori_loop(0, C // 16, inner, 0)
        lax.fori_loop(0, CH, body, 0)
        pltpu.sync_copy(o_v, out_hbm.at[pl.ds(base, CH)])


def _full(shape):
    return pl.BlockSpec(shape, lambda *_: tuple(0 for _ in shape))


def kernel(x, struct_embed, W_qkv, W_out, b_out, ln1_g, ln1_b, ln2_g, ln2_b,
           W_route, b_route, W_noise, b_noise, We1, be1, We2, be2, idx):
    B_, T_, C_ = x.shape
    xf = x.reshape(T_, C_)
    ids = idx.reshape(T_)
    sid = ((ids == 1) * 1 + (ids == 2) * 2 + (ids == 3) * 3).astype(I32)
    onehot = (sid[:, None] == jnp.arange(4)[None, :]).astype(F32)
    noise = jax.random.normal(jax.random.key(42), (T_, NE), dtype=F32)

    ln1_g2, ln1_b2 = ln1_g.reshape(1, C), ln1_b.reshape(1, C)
    ln2_g2, ln2_b2 = ln2_g.reshape(1, C), ln2_b.reshape(1, C)
    b_out2 = b_out.reshape(1, C)
    b_route2, b_noise2 = b_route.reshape(1, NE), b_noise.reshape(1, NE)

    h = pl.pallas_call(
        _prologue_kernel,
        grid=(NTB,),
        in_specs=[
            pl.BlockSpec((TB, C), lambda i: (i, 0)),
            pl.BlockSpec((TB, 4), lambda i: (i, 0)),
            _full((4, C)),
            _full((1, C)),
            _full((1, C)),
        ],
        out_specs=pl.BlockSpec((TB, C), lambda i: (i, 0)),
        out_shape=jax.ShapeDtypeStruct((T, C), F32),
    )(xf, onehot, struct_embed, ln1_g2, ln1_b2)

    Wh = W_qkv.reshape(C, 3 * NH, HD).transpose(1, 0, 2)
    qkv3 = pl.pallas_call(
        _qkv_kernel,
        grid=(3 * NH,),
        in_specs=[
            _full((T, C)),
            pl.BlockSpec((1, C, HD), lambda j: (j, 0, 0)),
        ],
        out_specs=pl.BlockSpec((1, T, HD), lambda j: (j, 0, 0)),
        out_shape=jax.ShapeDtypeStruct((3 * NH, T, HD), F32),
    )(h, Wh)

    ctx3 = pl.pallas_call(
        _attn_kernel,
        grid=(NH, NQB),
        in_specs=[
            pl.BlockSpec((1, QB, HD), lambda h, qb: (h, qb, 0)),
            pl.BlockSpec((1, T, HD), lambda h, qb: (NH + h, 0, 0)),
            pl.BlockSpec((1, T, HD), lambda h, qb: (2 * NH + h, 0, 0)),
        ],
        out_specs=pl.BlockSpec((1, QB, HD), lambda h, qb: (h, qb, 0)),
        out_shape=jax.ShapeDtypeStruct((NH, T, HD), F32),
    )(qkv3, qkv3, qkv3)
    ctx = ctx3.transpose(1, 0, 2).reshape(T, C)

    x2, m, noisy = pl.pallas_call(
        _post_kernel,
        grid=(NTB,),
        in_specs=[
            pl.BlockSpec((TB, C), lambda i: (i, 0)),
            pl.BlockSpec((TB, C), lambda i: (i, 0)),
            _full((C, C)),
            _full((1, C)),
            _full((1, C)),
            _full((1, C)),
            _full((C, NE)),
            _full((1, NE)),
            _full((C, NE)),
            _full((1, NE)),
            pl.BlockSpec((TB, NE), lambda i: (i, 0)),
        ],
        out_specs=[
            pl.BlockSpec((TB, C), lambda i: (i, 0)),
            pl.BlockSpec((TB, C), lambda i: (i, 0)),
            pl.BlockSpec((TB, NE), lambda i: (i, 0)),
        ],
        out_shape=[
            jax.ShapeDtypeStruct((T, C), F32),
            jax.ShapeDtypeStruct((T, C), F32),
            jax.ShapeDtypeStruct((T, NE), F32),
        ],
    )(xf, ctx, W_out, b_out2, ln2_g2, ln2_b2, W_route, b_route2,
      W_noise, b_noise2, noise)

    pos, gv, be_arr, nact = pl.pallas_call(
        _route_kernel,
        grid=(1,),
        in_specs=[_full((T, NE))],
        out_specs=[
            _full((A, 1)),
            _full((A, 128)),
            _full((NB, 1)),
            _full((1, 1)),
        ],
        out_shape=[
            jax.ShapeDtypeStruct((A, 1), I32),
            jax.ShapeDtypeStruct((A, 128), F32),
            jax.ShapeDtypeStruct((NB, 1), I32),
            jax.ShapeDtypeStruct((1, 1), I32),
        ],
    )(noisy)

    pos2d = pos.reshape(NW, A // NW)

    mesh = plsc.VectorSubcoreMesh(core_axis_name="c", subcore_axis_name="s")
    dispatch = pl.kernel(
        _dispatch_body, mesh=mesh,
        out_type=[
            jax.ShapeDtypeStruct((S, C), F32),
            jax.ShapeDtypeStruct((S, 128), F32),
        ],
        scratch_types=[
            pltpu.VMEM((A // NW,), I32),
            pltpu.VMEM((A // NW, C), F32),
            pltpu.VMEM((A // NW, 128), F32),
            pltpu.SemaphoreType.DMA,
        ],
    )
    Xs, Gs = dispatch(m, pos2d, gv)

    ys = pl.pallas_call(
        _ffn_kernel,
        grid_spec=pltpu.PrefetchScalarGridSpec(
            num_scalar_prefetch=2,
            grid=(NB,),
            in_specs=[
                pl.BlockSpec((G, C), lambda b, be, na: (b, 0)),
                pl.BlockSpec((1, C, DFF), lambda b, be, na: (be[b], 0, 0)),
                pl.BlockSpec((1, 1, DFF), lambda b, be, na: (be[b], 0, 0)),
                pl.BlockSpec((1, DFF, C), lambda b, be, na: (be[b], 0, 0)),
                pl.BlockSpec((1, 1, C), lambda b, be, na: (be[b], 0, 0)),
                pl.BlockSpec((G, 128), lambda b, be, na: (b, 0)),
            ],
            out_specs=pl.BlockSpec((G, C), lambda b, be, na: (b, 0)),
        ),
        out_shape=jax.ShapeDtypeStruct((S, C), F32),
        compiler_params=pltpu.CompilerParams(
            dimension_semantics=("arbitrary",)),
    )(be_arr.reshape(NB), nact.reshape(1), Xs, We1,
      be1.reshape(NE, 1, DFF), We2, be2.reshape(NE, 1, C), Gs)

    combine = pl.kernel(
        _combine_body, mesh=mesh,
        out_type=jax.ShapeDtypeStruct((T, C), F32),
        scratch_types=[
            pltpu.VMEM((32,), I32),
            pltpu.VMEM((32,), I32),
            pltpu.VMEM((32, C), F32),
            pltpu.VMEM((32, C), F32),
            pltpu.VMEM((32, C), F32),
            pltpu.SemaphoreType.DMA,
        ],
    )
    out = combine(x2, pos.reshape(A), ys)

    return out.reshape(B_, T_, C_)


# speedup vs baseline: 1.4826x; 1.0686x over previous
import functools

import jax
import jax.numpy as jnp
from jax import lax
from jax.experimental import pallas as pl
from jax.experimental.pallas import tpu as pltpu
from jax.experimental.pallas import tpu_sc as plsc

F32 = jnp.float32
BF16 = jnp.bfloat16
I32 = jnp.int32
T = 2048
C = 768
NH = 12
HD = 64
NE = 8
DFF = 3072
TB = 256
QB = 512
NTB = T // TB
NQB = T // QB
NP = NH // 2
A = 2 * T
G = 256
NB = A // G + NE
S = NB * G

NSC = 2
NTILE = 16
NW = NSC * NTILE


def _ln(x, g, b):
    mu = jnp.mean(x, axis=-1, keepdims=True)
    var = jnp.mean((x - mu) ** 2, axis=-1, keepdims=True)
    return (x - mu) / jnp.sqrt(var + 1e-5) * g + b


def _prologue_kernel(x_ref, oh_ref, se_ref, g_ref, b_ref, h_ref):
    x = x_ref[...]
    h = _ln(x, g_ref[...], b_ref[...])
    h_ref[...] = h + jnp.dot(oh_ref[...], se_ref[...],
                             preferred_element_type=F32)


def _qkv_kernel(h_ref, w_ref, qkv_ref):
    hb = h_ref[...].astype(BF16)
    qkv_ref[0] = jnp.dot(hb, w_ref[...],
                         preferred_element_type=F32).astype(BF16)


def _attn_kernel(q_ref, k_ref, v_ref, o_ref):
    q = q_ref[0]
    k = k_ref[0]
    v = v_ref[0]
    parts = []
    for p0 in (0, 1):
        sl = slice(p0 * HD, (p0 + 1) * HD)
        s = lax.dot_general(q[:, sl], k[:, sl], (((1,), (1,)), ((), ())),
                            preferred_element_type=F32) * 0.125
        s = jnp.clip(s, -30.0, 30.0)
        mx = jnp.max(s, axis=-1, keepdims=True)
        p = jnp.exp(s - mx)
        denom = jnp.sum(p, axis=-1, keepdims=True)
        parts.append(jnp.dot(p.astype(BF16), v[:, sl],
                             preferred_element_type=F32) / denom)
    o_ref[0] = jnp.concatenate(parts, axis=1)


def _post_kernel(x_ref, ctx_ref, wout_ref, bout_ref, g2_ref, b2_ref,
                 wr_ref, br_ref, wn_ref, bn_ref, nz_ref,
                 x2_ref, m_ref, noisy_ref):
    acc = bout_ref[...]
    for p in range(NP):
        acc = acc + jnp.dot(ctx_ref[p].astype(BF16), wout_ref[p],
                            preferred_element_type=F32)
    x2 = x_ref[...] + acc
    x2_ref[...] = x2
    m = _ln(x2, g2_ref[...], b2_ref[...])
    m_ref[...] = m
    logits = jnp.dot(m, wr_ref[...], preferred_element_type=F32) + br_ref[...]
    nl = jnp.dot(m, wn_ref[...], preferred_element_type=F32) + bn_ref[...]
    sp = jnp.maximum(nl, 0.0) + jnp.log1p(jnp.exp(-jnp.abs(nl)))
    noisy_ref[...] = logits + nz_ref[...] * sp


def _cumsum0(x):
    n = x.shape[0]
    k = 1
    while k < n:
        pad = jnp.zeros((k, x.shape[1]), x.dtype)
        x = x + jnp.concatenate([pad, x[:-k]], axis=0)
        k *= 2
    return x


def _route_kernel(noisy_ref, pos_ref, gv_ref, be_ref, nact_ref):
    noisy = noisy_ref[...]
    iota8 = lax.broadcasted_iota(I32, (T, NE), 1)
    v1 = jnp.max(noisy, axis=1, keepdims=True)
    is1 = noisy == v1
    e1 = jnp.min(jnp.where(is1, iota8, NE), axis=1, keepdims=True)
    masked = jnp.where(iota8 == e1, -jnp.inf, noisy)
    v2 = jnp.max(masked, axis=1, keepdims=True)
    is2 = masked == v2
    e2 = jnp.min(jnp.where(is2, iota8, NE), axis=1, keepdims=True)

    d = jnp.exp(v2 - v1)
    g1 = 1.0 / (1.0 + d)
    g2 = d / (1.0 + d)

    oh1 = (iota8 == e1).astype(I32)
    oh2 = (iota8 == e2).astype(I32)
    oh = jnp.concatenate([oh1, oh2], axis=0)
    cum = _cumsum0(oh)
    rank = jnp.sum(cum * oh, axis=1, keepdims=True) - 1

    counts = cum[A - 1:A, :]
    pcnt = ((counts + (G - 1)) // G) * G
    inc = pcnt
    for k in (1, 2, 4):
        inc = inc + jnp.concatenate(
            [jnp.zeros((1, k), I32), inc[:, :-k]], axis=1)
    poff = inc - pcnt

    po = jnp.sum(oh * poff, axis=1, keepdims=True)
    pos_ref[...] = po + rank

    g_all = jnp.concatenate([g1, g2], axis=0)
    gv_ref[...] = jnp.broadcast_to(g_all, (A, 128))

    iob = lax.broadcasted_iota(I32, (NB, NE), 0) * G
    be_ref[...] = jnp.sum((iob >= poff).astype(I32), axis=1,
                          keepdims=True) - 1
    nact_ref[...] = jnp.sum(pcnt, axis=1, keepdims=True) // G


def _ffn_kernel(be_ref, nact_ref, xs_ref, w1_ref, b1_ref, w2_ref, b2_ref,
                gs_ref, ys_ref):
    b = pl.program_id(0)

    @pl.when(b < nact_ref[0])
    def _():
        xb = xs_ref[...].astype(BF16)
        h = jnp.dot(xb, w1_ref[0], preferred_element_type=F32) + b1_ref[0]
        h = jax.nn.gelu(h)
        y = jnp.dot(h.astype(BF16), w2_ref[0],
                    preferred_element_type=F32) + b2_ref[0]
        gate = gs_ref[...][:, :1]
        ys_ref[...] = gate * y


def _dispatch_body(m_hbm, pos_hbm, gv_hbm, xs_hbm, gs_hbm,
                   idx_v, rows_v, gvr_v, sem):
    c = lax.axis_index("c")
    s = lax.axis_index("s")
    w = s * NSC + c
    tok = lax.rem(w, NTILE) * (T // NTILE)
    pltpu.sync_copy(pos_hbm.at[w], idx_v)
    pltpu.sync_copy(m_hbm.at[pl.ds(tok, A // NW)], rows_v)
    pltpu.sync_copy(gv_hbm.at[pl.ds(w * (A // NW), A // NW)], gvr_v)
    pltpu.async_copy(rows_v, xs_hbm.at[idx_v], sem).wait()
    pltpu.async_copy(gvr_v, gs_hbm.at[idx_v], sem).wait()


def _combine_body(x2_hbm, pos_hbm, ys_hbm, out_hbm,
                  i1_v, i2_v, y1_v, y2_v, o_v, sem):
    c = lax.axis_index("c")
    s = lax.axis_index("s")
    w = s * NSC + c
    CH = 32
    for half in range(2):
        base = w * 64 + half * CH
        pltpu.sync_copy(pos_hbm.at[pl.ds(base, CH)], i1_v)
        pltpu.sync_copy(pos_hbm.at[pl.ds(T + base, CH)], i2_v)
        pltpu.async_copy(ys_hbm.at[i1_v], y1_v, sem).wait()
        pltpu.async_copy(ys_hbm.at[i2_v], y2_v, sem).wait()
        pltpu.sync_copy(x2_hbm.at[pl.ds(base, CH)], o_v)

        def body(i, _):
            for j in range(C // 16):
                sl = pl.ds(j * 16, 16)
                o_v[i, sl] = o_v[i, sl] + y1_v[i, sl] + y2_v[i, sl]
            return 0
        lax.fori_loop(0, CH, body, 0)
        pltpu.sync_copy(o_v, out_hbm.at[pl.ds(base, CH)])


def _full(shape):
    return pl.BlockSpec(shape, lambda *_: tuple(0 for _ in shape))


def kernel(x, struct_embed, W_qkv, W_out, b_out, ln1_g, ln1_b, ln2_g, ln2_b,
           W_route, b_route, W_noise, b_noise, We1, be1, We2, be2, idx):
    B_, T_, C_ = x.shape
    xf = x.reshape(T_, C_)
    ids = idx.reshape(T_)
    sid = ((ids == 1) * 1 + (ids == 2) * 2 + (ids == 3) * 3).astype(I32)
    onehot = (sid[:, None] == jnp.arange(4)[None, :]).astype(F32)
    noise = jax.random.normal(jax.random.key(42), (T_, NE), dtype=F32)

    ln1_g2, ln1_b2 = ln1_g.reshape(1, C), ln1_b.reshape(1, C)
    ln2_g2, ln2_b2 = ln2_g.reshape(1, C), ln2_b.reshape(1, C)
    b_out2 = b_out.reshape(1, C)
    b_route2, b_noise2 = b_route.reshape(1, NE), b_noise.reshape(1, NE)

    h = pl.pallas_call(
        _prologue_kernel,
        grid=(NTB,),
        in_specs=[
            pl.BlockSpec((TB, C), lambda i: (i, 0)),
            pl.BlockSpec((TB, 4), lambda i: (i, 0)),
            _full((4, C)),
            _full((1, C)),
            _full((1, C)),
        ],
        out_specs=pl.BlockSpec((TB, C), lambda i: (i, 0)),
        out_shape=jax.ShapeDtypeStruct((T, C), F32),
    )(xf, onehot, struct_embed, ln1_g2, ln1_b2)

    qkv2 = pl.pallas_call(
        _qkv_kernel,
        grid=(3 * NP,),
        in_specs=[
            _full((T, C)),
            pl.BlockSpec((C, 128), lambda j: (0, j)),
        ],
        out_specs=pl.BlockSpec((1, T, 128), lambda j: (j, 0, 0)),
        out_shape=jax.ShapeDtypeStruct((3 * NP, T, 128), BF16),
    )(h, W_qkv.astype(BF16))

    ctx2 = pl.pallas_call(
        _attn_kernel,
        grid=(NP, NQB),
        in_specs=[
            pl.BlockSpec((1, QB, 128), lambda h, qb: (h, qb, 0)),
            pl.BlockSpec((1, T, 128), lambda h, qb: (NP + h, 0, 0)),
            pl.BlockSpec((1, T, 128), lambda h, qb: (2 * NP + h, 0, 0)),
        ],
        out_specs=pl.BlockSpec((1, QB, 128), lambda h, qb: (h, qb, 0)),
        out_shape=jax.ShapeDtypeStruct((NP, T, 128), F32),
    )(qkv2, qkv2, qkv2)

    x2, m, noisy = pl.pallas_call(
        _post_kernel,
        grid=(NTB,),
        in_specs=[
            pl.BlockSpec((TB, C), lambda i: (i, 0)),
            pl.BlockSpec((NP, TB, 128), lambda i: (0, i, 0)),
            _full((NP, 128, C)),
            _full((1, C)),
            _full((1, C)),
            _full((1, C)),
            _full((C, NE)),
            _full((1, NE)),
            _full((C, NE)),
            _full((1, NE)),
            pl.BlockSpec((TB, NE), lambda i: (i, 0)),
        ],
        out_specs=[
            pl.BlockSpec((TB, C), lambda i: (i, 0)),
            pl.BlockSpec((TB, C), lambda i: (i, 0)),
            pl.BlockSpec((TB, NE), lambda i: (i, 0)),
        ],
        out_shape=[
            jax.ShapeDtypeStruct((T, C), F32),
            jax.ShapeDtypeStruct((T, C), F32),
            jax.ShapeDtypeStruct((T, NE), F32),
        ],
    )(xf, ctx2, W_out.reshape(NP, 128, C).astype(BF16), b_out2, ln2_g2,
      ln2_b2, W_route, b_route2, W_noise, b_noise2, noise)

    pos, gv, be_arr, nact = pl.pallas_call(
        _route_kernel,
        grid=(1,),
        in_specs=[_full((T, NE))],
        out_specs=[
            _full((A, 1)),
            _full((A, 128)),
            _full((NB, 1)),
            _full((1, 1)),
        ],
        out_shape=[
            jax.ShapeDtypeStruct((A, 1), I32),
            jax.ShapeDtypeStruct((A, 128), F32),
            jax.ShapeDtypeStruct((NB, 1), I32),
            jax.ShapeDtypeStruct((1, 1), I32),
        ],
    )(noisy)

    pos2d = pos.reshape(NW, A // NW)

    mesh = plsc.VectorSubcoreMesh(core_axis_name="c", subcore_axis_name="s")
    dispatch = pl.kernel(
        _dispatch_body, mesh=mesh,
        out_type=[
            jax.ShapeDtypeStruct((S, C), F32),
            jax.ShapeDtypeStruct((S, 128), F32),
        ],
        scratch_types=[
            pltpu.VMEM((A // NW,), I32),
            pltpu.VMEM((A // NW, C), F32),
            pltpu.VMEM((A // NW, 128), F32),
            pltpu.SemaphoreType.DMA,
        ],
    )
    Xs, Gs = dispatch(m, pos2d, gv)

    ys = pl.pallas_call(
        _ffn_kernel,
        grid_spec=pltpu.PrefetchScalarGridSpec(
            num_scalar_prefetch=2,
            grid=(NB,),
            in_specs=[
                pl.BlockSpec((G, C), lambda b, be, na: (b, 0)),
                pl.BlockSpec((1, C, DFF), lambda b, be, na: (be[b], 0, 0)),
                pl.BlockSpec((1, 1, DFF), lambda b, be, na: (be[b], 0, 0)),
                pl.BlockSpec((1, DFF, C), lambda b, be, na: (be[b], 0, 0)),
                pl.BlockSpec((1, 1, C), lambda b, be, na: (be[b], 0, 0)),
                pl.BlockSpec((G, 128), lambda b, be, na: (b, 0)),
            ],
            out_specs=pl.BlockSpec((G, C), lambda b, be, na: (b, 0)),
        ),
        out_shape=jax.ShapeDtypeStruct((S, C), F32),
        compiler_params=pltpu.CompilerParams(
            dimension_semantics=("arbitrary",)),
    )(be_arr.reshape(NB), nact.reshape(1), Xs, We1.astype(BF16),
      be1.reshape(NE, 1, DFF), We2.astype(BF16), be2.reshape(NE, 1, C), Gs)

    combine = pl.kernel(
        _combine_body, mesh=mesh,
        out_type=jax.ShapeDtypeStruct((T, C), F32),
        scratch_types=[
            pltpu.VMEM((32,), I32),
            pltpu.VMEM((32,), I32),
            pltpu.VMEM((32, C), F32),
            pltpu.VMEM((32, C), F32),
            pltpu.VMEM((32, C), F32),
            pltpu.SemaphoreType.DMA,
        ],
    )
    out = combine(x2, pos.reshape(A), ys)

    return out.reshape(B_, T_, C_)


# speedup vs baseline: 1.5838x; 1.0683x over previous
import functools

import jax
import jax.numpy as jnp
from jax import lax
from jax.experimental import pallas as pl
from jax.experimental.pallas import tpu as pltpu
from jax.experimental.pallas import tpu_sc as plsc

F32 = jnp.float32
BF16 = jnp.bfloat16
I32 = jnp.int32
T = 2048
C = 768
NH = 12
HD = 64
NE = 8
DFF = 3072
TB = 256
QB = 512
NTB = T // TB
NQB = T // QB
NP = NH // 2
A = 2 * T
G = 256
NB = A // G + NE
S = NB * G

NSC = 2
NTILE = 16
NW = NSC * NTILE
CH = 32


def _ln(x, g, b):
    mu = jnp.mean(x, axis=-1, keepdims=True)
    var = jnp.mean((x - mu) ** 2, axis=-1, keepdims=True)
    return (x - mu) / jnp.sqrt(var + 1e-5) * g + b


def _prologue_kernel(x_ref, oh_ref, se_ref, g_ref, b_ref, wq_ref, qkv_ref):
    x = x_ref[...]
    h = _ln(x, g_ref[...], b_ref[...])
    h = h + jnp.dot(oh_ref[...], se_ref[...], preferred_element_type=F32)
    qkv_ref[...] = jnp.dot(h.astype(BF16), wq_ref[...],
                           preferred_element_type=F32).astype(BF16)


def _attn_kernel(q_ref, k_ref, v_ref, o_ref):
    q = q_ref[...]
    k = k_ref[...]
    v = v_ref[...]
    parts = []
    for p0 in (0, 1):
        sl = slice(p0 * HD, (p0 + 1) * HD)
        s = lax.dot_general(q[:, sl], k[:, sl], (((1,), (1,)), ((), ())),
                            preferred_element_type=F32) * 0.125
        s = jnp.clip(s, -30.0, 30.0)
        mx = jnp.max(s, axis=-1, keepdims=True)
        p = jnp.exp(s - mx)
        denom = jnp.sum(p, axis=-1, keepdims=True)
        parts.append(jnp.dot(p.astype(BF16), v[:, sl],
                             preferred_element_type=F32) / denom)
    o_ref[...] = jnp.concatenate(parts, axis=1)


def _post_kernel(x_ref, ctx_ref, wout_ref, bout_ref, g2_ref, b2_ref,
                 wr_ref, br_ref, wn_ref, bn_ref, nz_ref,
                 x2_ref, m_ref, noisy_ref):
    x2 = x_ref[...] + jnp.dot(ctx_ref[...].astype(BF16), wout_ref[...],
                              preferred_element_type=F32) + bout_ref[...]
    x2_ref[...] = x2
    m = _ln(x2, g2_ref[...], b2_ref[...])
    m_ref[...] = m
    logits = jnp.dot(m, wr_ref[...], preferred_element_type=F32) + br_ref[...]
    nl = jnp.dot(m, wn_ref[...], preferred_element_type=F32) + bn_ref[...]
    sp = jnp.maximum(nl, 0.0) + jnp.log1p(jnp.exp(-jnp.abs(nl)))
    noisy_ref[...] = logits + nz_ref[...] * sp


def _cumsum0(x):
    n = x.shape[0]
    k = 1
    while k < n:
        pad = jnp.zeros((k, x.shape[1]), x.dtype)
        x = x + jnp.concatenate([pad, x[:-k]], axis=0)
        k *= 2
    return x


def _route_kernel(noisy_ref, pos_ref, gv_ref, be_ref, nact_ref):
    noisy = noisy_ref[...]
    iota8 = lax.broadcasted_iota(I32, (T, NE), 1)
    v1 = jnp.max(noisy, axis=1, keepdims=True)
    is1 = noisy == v1
    e1 = jnp.min(jnp.where(is1, iota8, NE), axis=1, keepdims=True)
    masked = jnp.where(iota8 == e1, -jnp.inf, noisy)
    v2 = jnp.max(masked, axis=1, keepdims=True)
    is2 = masked == v2
    e2 = jnp.min(jnp.where(is2, iota8, NE), axis=1, keepdims=True)

    d = jnp.exp(v2 - v1)
    g1 = 1.0 / (1.0 + d)
    g2 = d / (1.0 + d)

    oh1 = (iota8 == e1).astype(I32)
    oh2 = (iota8 == e2).astype(I32)
    oh = jnp.concatenate([oh1, oh2], axis=0)
    cum = _cumsum0(oh)
    rank = jnp.sum(cum * oh, axis=1, keepdims=True) - 1

    counts = cum[A - 1:A, :]
    pcnt = ((counts + (G - 1)) // G) * G
    inc = pcnt
    for k in (1, 2, 4):
        inc = inc + jnp.concatenate(
            [jnp.zeros((1, k), I32), inc[:, :-k]], axis=1)
    poff = inc - pcnt

    po = jnp.sum(oh * poff, axis=1, keepdims=True)
    pos_ref[...] = po + rank

    g_all = jnp.concatenate([g1, g2], axis=0)
    gv_ref[...] = jnp.broadcast_to(g_all, (A, 128))

    iob = lax.broadcasted_iota(I32, (NB, NE), 0) * G
    be_ref[...] = jnp.sum((iob >= poff).astype(I32), axis=1,
                          keepdims=True) - 1
    nact_ref[...] = jnp.sum(pcnt, axis=1, keepdims=True) // G


def _ffn_kernel(be_ref, nact_ref, xs_ref, w1_ref, b1_ref, w2_ref, b2_ref,
                ys_ref):
    b = pl.program_id(0)

    @pl.when(b < nact_ref[0])
    def _():
        xb = xs_ref[...].astype(BF16)
        h = jnp.dot(xb, w1_ref[0], preferred_element_type=F32) + b1_ref[0]
        h = jax.nn.gelu(h)
        ys_ref[...] = jnp.dot(h.astype(BF16), w2_ref[0],
                              preferred_element_type=F32) + b2_ref[0]


def _dispatch_body(m_hbm, pos_hbm, xs_hbm, idx_v, rows_v, sem_i, sem_r, sem):
    c = lax.axis_index("c")
    s = lax.axis_index("s")
    w = s * NSC + c
    tok = lax.rem(w, NTILE) * (T // NTILE)
    pltpu.sync_copy(pos_hbm.at[pl.ds(w * (A // NW), A // NW)], idx_v)
    pltpu.sync_copy(m_hbm.at[pl.ds(tok, A // NW)], rows_v)
    pltpu.async_copy(rows_v, xs_hbm.at[idx_v], sem).wait()


def _combine_body(x2_hbm, pos_hbm, g_hbm, ys_hbm, out_hbm,
                  i1_v, i2_v, g1_v, g2_v, y1_v, y2_v, o_v,
                  sem_i, sem_g, sem_x, sem_y):
    c = lax.axis_index("c")
    s = lax.axis_index("s")
    w = s * NSC + c
    for half in range(2):
        base = w * 64 + half * CH
        ci1 = pltpu.async_copy(pos_hbm.at[pl.ds(base, CH)], i1_v, sem_i)
        ci2 = pltpu.async_copy(pos_hbm.at[pl.ds(T + base, CH)], i2_v, sem_i)
        cg1 = pltpu.async_copy(g_hbm.at[pl.ds(base, CH)], g1_v, sem_g)
        cg2 = pltpu.async_copy(g_hbm.at[pl.ds(T + base, CH)], g2_v, sem_g)
        cx = pltpu.async_copy(x2_hbm.at[pl.ds(base, CH)], o_v, sem_x)
        ci1.wait()
        ci2.wait()
        cy1 = pltpu.async_copy(ys_hbm.at[i1_v], y1_v, sem_y)
        cy2 = pltpu.async_copy(ys_hbm.at[i2_v], y2_v, sem_y)
        cg1.wait()
        cg2.wait()
        cx.wait()
        cy1.wait()
        cy2.wait()

        def body(i, _):
            ga = g1_v[i, pl.ds(0, 16)]
            gb = g2_v[i, pl.ds(0, 16)]
            for j in range(C // 16):
                sl = pl.ds(j * 16, 16)
                o_v[i, sl] = o_v[i, sl] + y1_v[i, sl] * ga + y2_v[i, sl] * gb
            return 0
        lax.fori_loop(0, CH, body, 0)
        pltpu.sync_copy(o_v, out_hbm.at[pl.ds(base, CH)])


def _full(shape):
    return pl.BlockSpec(shape, lambda *_: tuple(0 for _ in shape))


def kernel(x, struct_embed, W_qkv, W_out, b_out, ln1_g, ln1_b, ln2_g, ln2_b,
           W_route, b_route, W_noise, b_noise, We1, be1, We2, be2, idx):
    B_, T_, C_ = x.shape
    xf = x.reshape(T_, C_)
    ids = idx.reshape(T_)
    sid = ((ids == 1) * 1 + (ids == 2) * 2 + (ids == 3) * 3).astype(I32)
    onehot = (sid[:, None] == jnp.arange(4)[None, :]).astype(F32)
    noise = jax.random.normal(jax.random.key(42), (T_, NE), dtype=F32)

    ln1_g2, ln1_b2 = ln1_g.reshape(1, C), ln1_b.reshape(1, C)
    ln2_g2, ln2_b2 = ln2_g.reshape(1, C), ln2_b.reshape(1, C)
    b_out2 = b_out.reshape(1, C)
    b_route2, b_noise2 = b_route.reshape(1, NE), b_noise.reshape(1, NE)

    qkv = pl.pallas_call(
        _prologue_kernel,
        grid=(NTB,),
        in_specs=[
            pl.BlockSpec((TB, C), lambda i: (i, 0)),
            pl.BlockSpec((TB, 4), lambda i: (i, 0)),
            _full((4, C)),
            _full((1, C)),
            _full((1, C)),
            _full((C, 3 * C)),
        ],
        out_specs=pl.BlockSpec((TB, 3 * C), lambda i: (i, 0)),
        out_shape=jax.ShapeDtypeStruct((T, 3 * C), BF16),
    )(xf, onehot, struct_embed, ln1_g2, ln1_b2, W_qkv.astype(BF16))

    ctx = pl.pallas_call(
        _attn_kernel,
        grid=(NP, NQB),
        in_specs=[
            pl.BlockSpec((QB, 128), lambda h, qb: (qb, h)),
            pl.BlockSpec((T, 128), lambda h, qb: (0, NP + h)),
            pl.BlockSpec((T, 128), lambda h, qb: (0, 2 * NP + h)),
        ],
        out_specs=pl.BlockSpec((QB, 128), lambda h, qb: (qb, h)),
        out_shape=jax.ShapeDtypeStruct((T, C), F32),
    )(qkv, qkv, qkv)

    x2, m, noisy = pl.pallas_call(
        _post_kernel,
        grid=(NTB,),
        in_specs=[
            pl.BlockSpec((TB, C), lambda i: (i, 0)),
            pl.BlockSpec((TB, C), lambda i: (i, 0)),
            _full((C, C)),
            _full((1, C)),
            _full((1, C)),
            _full((1, C)),
            _full((C, NE)),
            _full((1, NE)),
            _full((C, NE)),
            _full((1, NE)),
            pl.BlockSpec((TB, NE), lambda i: (i, 0)),
        ],
        out_specs=[
            pl.BlockSpec((TB, C), lambda i: (i, 0)),
            pl.BlockSpec((TB, C), lambda i: (i, 0)),
            pl.BlockSpec((TB, NE), lambda i: (i, 0)),
        ],
        out_shape=[
            jax.ShapeDtypeStruct((T, C), F32),
            jax.ShapeDtypeStruct((T, C), F32),
            jax.ShapeDtypeStruct((T, NE), F32),
        ],
    )(xf, ctx, W_out.astype(BF16), b_out2, ln2_g2, ln2_b2, W_route,
      b_route2, W_noise, b_noise2, noise)

    pos, gv, be_arr, nact = pl.pallas_call(
        _route_kernel,
        grid=(1,),
        in_specs=[_full((T, NE))],
        out_specs=[
            _full((A, 1)),
            _full((A, 128)),
            _full((NB, 1)),
            _full((1, 1)),
        ],
        out_shape=[
            jax.ShapeDtypeStruct((A, 1), I32),
            jax.ShapeDtypeStruct((A, 128), F32),
            jax.ShapeDtypeStruct((NB, 1), I32),
            jax.ShapeDtypeStruct((1, 1), I32),
        ],
    )(noisy)

    posf = pos.reshape(A)

    mesh = plsc.VectorSubcoreMesh(core_axis_name="c", subcore_axis_name="s")
    dispatch = pl.kernel(
        _dispatch_body, mesh=mesh,
        out_type=jax.ShapeDtypeStruct((S, C), F32),
        scratch_types=[
            pltpu.VMEM((A // NW,), I32),
            pltpu.VMEM((A // NW, C), F32),
            pltpu.SemaphoreType.DMA,
            pltpu.SemaphoreType.DMA,
            pltpu.SemaphoreType.DMA,
        ],
    )
    Xs = dispatch(m, posf)

    ys = pl.pallas_call(
        _ffn_kernel,
        grid_spec=pltpu.PrefetchScalarGridSpec(
            num_scalar_prefetch=2,
            grid=(NB,),
            in_specs=[
                pl.BlockSpec((G, C), lambda b, be, na: (b, 0)),
                pl.BlockSpec((1, C, DFF), lambda b, be, na: (be[b], 0, 0)),
                pl.BlockSpec((1, 1, DFF), lambda b, be, na: (be[b], 0, 0)),
                pl.BlockSpec((1, DFF, C), lambda b, be, na: (be[b], 0, 0)),
                pl.BlockSpec((1, 1, C), lambda b, be, na: (be[b], 0, 0)),
            ],
            out_specs=pl.BlockSpec((G, C), lambda b, be, na: (b, 0)),
        ),
        out_shape=jax.ShapeDtypeStruct((S, C), F32),
        compiler_params=pltpu.CompilerParams(
            dimension_semantics=("arbitrary",)),
    )(be_arr.reshape(NB), nact.reshape(1), Xs, We1.astype(BF16),
      be1.reshape(NE, 1, DFF), We2.astype(BF16), be2.reshape(NE, 1, C))

    combine = pl.kernel(
        _combine_body, mesh=mesh,
        out_type=jax.ShapeDtypeStruct((T, C), F32),
        scratch_types=[
            pltpu.VMEM((CH,), I32),
            pltpu.VMEM((CH,), I32),
            pltpu.VMEM((CH, 128), F32),
            pltpu.VMEM((CH, 128), F32),
            pltpu.VMEM((CH, C), F32),
            pltpu.VMEM((CH, C), F32),
            pltpu.VMEM((CH, C), F32),
            pltpu.SemaphoreType.DMA,
            pltpu.SemaphoreType.DMA,
            pltpu.SemaphoreType.DMA,
            pltpu.SemaphoreType.DMA,
        ],
    )
    out = combine(x2, posf, gv, ys)

    return out.reshape(B_, T_, C_)
